# baseline (speedup 1.0000x reference)
import functools

import jax
import jax.numpy as jnp
from jax import lax
from jax.experimental import pallas as pl
from jax.experimental.pallas import tpu as pltpu
from jax.experimental.pallas import tpu_sc as plsc

T = 16384
D = 64

_info = plsc.get_sparse_core_info()
_NC, _NS = _info.num_cores, _info.num_subcores
_NW = _NC * _NS
_BPW = T // _NW
_CHUNK = 128
_NCHUNK = _BPW // _CHUNK

_mesh = plsc.VectorSubcoreMesh(core_axis_name="c", subcore_axis_name="s")


@functools.partial(
    pl.kernel,
    mesh=_mesh,
    out_type=jax.ShapeDtypeStruct((T, D), jnp.float32),
    scratch_types=[
        pltpu.VMEM((_NCHUNK, _CHUNK), jnp.int32),
        pltpu.VMEM((_BPW, D), jnp.float32),
        pltpu.SemaphoreType.DMA,
    ],
    compiler_params=pltpu.CompilerParams(use_tc_tiling_on_sc=False),
)
def _gather_kernel(idx_hbm, table_hbm, out_hbm, idx_v, rows_v, sem):
    wid = lax.axis_index("s") * _NC + lax.axis_index("c")
    pltpu.sync_copy(idx_hbm.at[wid], idx_v)
    copies = []
    for j in range(_NCHUNK):
        copies.append(
            pltpu.async_copy(
                table_hbm.at[idx_v.at[j]],
                rows_v.at[pl.ds(j * _CHUNK, _CHUNK)],
                sem,
            )
        )
    for c in copies:
        c.wait()
    pltpu.sync_copy(rows_v, out_hbm.at[pl.ds(wid * _BPW, _BPW)])


def kernel(x, table):
    idx = x.astype(jnp.int32).reshape(_NW, _NCHUNK, _CHUNK)
    return _gather_kernel(idx, table)


# speedup vs baseline: 1.0265x; 1.0265x over previous
import functools

import jax
import jax.numpy as jnp
from jax import lax
from jax.experimental import pallas as pl
from jax.experimental.pallas import tpu as pltpu
from jax.experimental.pallas import tpu_sc as plsc

T = 16384
D = 64

_info = plsc.get_sparse_core_info()
_NC, _NS = _info.num_cores, _info.num_subcores
_NW = _NC * _NS
_BPW = T // _NW
_UNROLL = 16

_mesh = plsc.VectorSubcoreMesh(core_axis_name="c", subcore_axis_name="s")


@functools.partial(
    pl.kernel,
    mesh=_mesh,
    out_type=jax.ShapeDtypeStruct((T, D), jnp.float32),
    scratch_types=[
        pltpu.VMEM((_BPW,), jnp.int32),
        pltpu.SemaphoreType.DMA,
    ],
)
def _gather_kernel(idx_hbm, table_hbm, out_hbm, idx_v, sem):
    wid = lax.axis_index("s") * _NC + lax.axis_index("c")
    base = wid * _BPW
    pltpu.sync_copy(idx_hbm.at[wid], idx_v)

    @pl.loop(0, _BPW, step=_UNROLL)
    def _fire(c):
        vec = idx_v[pl.ds(c, _UNROLL)]
        for j in range(_UNROLL):
            row = vec[j]
            pltpu.async_copy(
                table_hbm.at[pl.ds(row, 1), :],
                out_hbm.at[pl.ds(base + c + j, 1), :],
                sem,
            )

    @pl.loop(0, _BPW, step=_UNROLL)
    def _drain(c):
        for j in range(_UNROLL):
            pltpu.make_async_copy(
                table_hbm.at[pl.ds(0, 1), :],
                out_hbm.at[pl.ds(base, 1), :],
                sem,
            ).wait()


def kernel(x, table):
    idx = x.astype(jnp.int32).reshape(_NW, _BPW)
    return _gather_kernel(idx, table)


# speedup vs baseline: 1.7066x; 1.6624x over previous
import functools

import jax
import jax.numpy as jnp
from jax import lax
from jax.experimental import pallas as pl
from jax.experimental.pallas import tpu as pltpu
from jax.experimental.pallas import tpu_sc as plsc

T = 16384
D = 64

_info = plsc.get_sparse_core_info()
_NC, _NS = _info.num_cores, _info.num_subcores
_NW = _NC * _NS
_BPW = T // _NW
_UNROLL = 16

_mesh = plsc.VectorSubcoreMesh(core_axis_name="c", subcore_axis_name="s")


@functools.partial(
    pl.kernel,
    mesh=_mesh,
    out_type=jax.ShapeDtypeStruct((T, D), jnp.float32),
    scratch_types=[
        pltpu.VMEM((_BPW,), jnp.int32),
        pltpu.VMEM((_BPW, D), jnp.float32),
        pltpu.SemaphoreType.DMA,
    ],
)
def _gather_kernel(idx_hbm, table_hbm, out_hbm, idx_v, rows_v, sem):
    wid = lax.axis_index("s") * _NC + lax.axis_index("c")
    base = wid * _BPW
    pltpu.sync_copy(idx_hbm.at[wid], idx_v)

    @pl.loop(0, _BPW, step=_UNROLL)
    def _fire(c):
        vec = idx_v[pl.ds(c, _UNROLL)]
        for j in range(_UNROLL):
            row = vec[j]
            pltpu.async_copy(
                table_hbm.at[pl.ds(row, 1), :],
                rows_v.at[pl.ds(c + j, 1), :],
                sem,
            )

    @pl.loop(0, _BPW, step=_UNROLL)
    def _drain(c):
        for j in range(_UNROLL):
            pltpu.make_async_copy(
                table_hbm.at[pl.ds(0, 1), :],
                rows_v.at[pl.ds(0, 1), :],
                sem,
            ).wait()

    pltpu.sync_copy(rows_v, out_hbm.at[pl.ds(base, _BPW)])


def kernel(x, table):
    idx = x.astype(jnp.int32).reshape(_NW, _BPW)
    return _gather_kernel(idx, table)


# speedup vs baseline: 2.9185x; 1.7102x over previous
import functools

import jax
import jax.numpy as jnp
from jax import lax
from jax.experimental import pallas as pl
from jax.experimental.pallas import tpu as pltpu
from jax.experimental.pallas import tpu_sc as plsc

T = 16384
D = 64
V = 1000000

_info = plsc.get_sparse_core_info()
_NC, _NS = _info.num_cores, _info.num_subcores
_NW = _NC * _NS
_L = 16
_CW = 256
_NBC = _CW // 128
_NBLK = (V + 127) // 128
_MAXOFF = _NBLK * 128 - _CW
_WIN = 2048
_RING = 8

_mesh = plsc.VectorSubcoreMesh(core_axis_name="c", subcore_axis_name="s")


@functools.partial(
    pl.kernel,
    mesh=_mesh,
    out_type=jax.ShapeDtypeStruct((T, D), jnp.float32),
    scratch_types=[
        pltpu.VMEM((T,), jnp.int32),
        pltpu.VMEM((T + _L,), jnp.int32),
        pltpu.VMEM((T + _L,), jnp.int32),
        pltpu.VMEM((_WIN + _L,), jnp.int32),
        pltpu.VMEM((_WIN + _L,), jnp.int32),
        pltpu.VMEM((D, _CW), jnp.float32),
        pltpu.VMEM((D, _CW), jnp.float32),
        pltpu.VMEM((_RING * _L, D), jnp.float32),
        pltpu.SemaphoreType.DMA,
        pltpu.SemaphoreType.DMA,
        pltpu.SemaphoreType.DMA,
    ],
    compiler_params=pltpu.CompilerParams(needs_layout_passes=False),
)
def _gather_kernel(idx_hbm, tab_hbm, out_hbm, idx_v, cand_r, cand_p,
                   sel_r, sel_p, buf_a, buf_b, tok_v, sem_a, sem_b, osem):
    wid = lax.axis_index("s") * _NC + lax.axis_index("c")
    lane = lax.iota(jnp.int32, _L)

    bs = wid * _NBLK // _NW
    be = (wid + 1) * _NBLK // _NW
    lo = bs * 128
    hi = be * 128

    pltpu.sync_copy(idx_hbm, idx_v)

    @pl.loop(0, T // _L, init_carry=0)
    def _filter(g, cnt):
        rows = idx_v[pl.ds(g * _L, _L)]
        m = (rows >= lo) & (rows < hi)
        ps = plsc.cumsum(m.astype(jnp.int32))
        offs = jnp.where(m, cnt + ps - 1, T + lane)
        plsc.store_scatter(cand_r, [offs], rows)
        plsc.store_scatter(cand_p, [offs], lane + g * _L)
        return cnt + ps[_L - 1]

    cnt = _filter

    def fetch(blk, buf, sem):
        off = jnp.minimum(blk * 128, _MAXOFF)
        off = pl.multiple_of(off, 128)
        pltpu.async_copy(tab_hbm.at[:, pl.ds(off, _CW)], buf, sem)

    def wait_fetch(buf, sem):
        pltpu.make_async_copy(tab_hbm.at[:, pl.ds(0, _CW)], buf, sem).wait()

    def process(blk, buf, kgrp):
        clo = blk * 128
        chi = jnp.minimum(blk + _NBC, be) * 128
        off = jnp.minimum(clo, _MAXOFF)

        @pl.loop(0, (cnt + _WIN - 1) // _WIN, init_carry=kgrp)
        def _window(w, kgrp):
            wbase = w * _WIN
            wlim = jnp.minimum(cnt - wbase, _WIN)

            @pl.loop(0, (wlim + _L - 1) // _L, init_carry=0)
            def _select(gi, scnt):
                rows = cand_r[pl.ds(wbase + gi * _L, _L)]
                pos = cand_p[pl.ds(wbase + gi * _L, _L)]
                m = (rows >= clo) & (rows < chi) & (lane + gi * _L < wlim)
                ps = plsc.cumsum(m.astype(jnp.int32))
                offs = jnp.where(m, scnt + ps - 1, _WIN + lane)
                plsc.store_scatter(sel_r, [offs], rows)
                plsc.store_scatter(sel_p, [offs], pos)
                return scnt + ps[_L - 1]

            scnt = _select

            @pl.loop(0, (scnt + _L - 1) // _L, init_carry=kgrp)
            def _groups(t, kgrp):
                rows = sel_r[pl.ds(t * _L, _L)]
                pos = sel_p[pl.ds(t * _L, _L)]
                m = lane < scnt - t * _L
                rows = jnp.where(m, rows, jnp.full((_L,), 1, jnp.int32) * rows[0])
                pos = jnp.where(m, pos, jnp.full((_L,), 1, jnp.int32) * pos[0])
                q = rows - off
                slot = lax.rem(kgrp, jnp.int32(_RING))

                @pl.when(kgrp >= _RING)
                def _():
                    @pl.loop(0, _L)
                    def _dr(j):
                        pltpu.make_async_copy(
                            tok_v.at[pl.ds(0, 1), :],
                            out_hbm.at[pl.ds(0, 1), :],
                            osem,
                        ).wait()

                srow = slot * _L + lane

                @pl.loop(0, D, unroll=8)
                def _cols(c):
                    vals = plsc.load_gather(
                        buf, [jnp.full((_L,), 1, jnp.int32) * c, q])
                    plsc.store_scatter(
                        tok_v, [srow, jnp.full((_L,), 1, jnp.int32) * c], vals)

                for jj in range(_L):
                    p = pos[jj]
                    pltpu.async_copy(
                        tok_v.at[pl.ds(slot * _L + jj, 1), :],
                        out_hbm.at[pl.ds(p, 1), :],
                        osem,
                    )
                return kgrp + 1

            return _groups

        return _window

    fetch(bs, buf_a, sem_a)

    @pl.loop(bs, be, step=2 * _NBC, init_carry=0)
    def _scan(j, kgrp):
        fetch(j + _NBC, buf_b, sem_b)
        wait_fetch(buf_a, sem_a)
        kgrp = process(j, buf_a, kgrp)
        fetch(j + 2 * _NBC, buf_a, sem_a)
        wait_fetch(buf_b, sem_b)
        kgrp = process(j + _NBC, buf_b, kgrp)
        return kgrp

    kfin = _scan
    wait_fetch(buf_a, sem_a)

    @pl.loop(0, jnp.minimum(kfin, _RING) * _L)
    def _drain(j):
        pltpu.make_async_copy(
            tok_v.at[pl.ds(0, 1), :],
            out_hbm.at[pl.ds(0, 1), :],
            osem,
        ).wait()


def kernel(x, table):
    idx = x.astype(jnp.int32)
    return _gather_kernel(idx, table.T)


# speedup vs baseline: 3.9020x; 1.3370x over previous
import functools

import jax
import jax.numpy as jnp
from jax import lax
from jax.experimental import pallas as pl
from jax.experimental.pallas import tpu as pltpu
from jax.experimental.pallas import tpu_sc as plsc

T = 16384
D = 64
V = 1000000

_info = plsc.get_sparse_core_info()
_NC, _NS = _info.num_cores, _info.num_subcores
_NW = _NC * _NS
_L = 16
_CW = 384
_NBC = _CW // 128
_NBLK = (V + 127) // 128
_MAXOFF = _NBLK * 128 - _CW
_WIN = 2048
_RING = 8

_mesh = plsc.VectorSubcoreMesh(core_axis_name="c", subcore_axis_name="s")


@functools.partial(
    pl.kernel,
    mesh=_mesh,
    out_type=jax.ShapeDtypeStruct((T, D), jnp.float32),
    scratch_types=[
        pltpu.VMEM((T,), jnp.int32),
        pltpu.VMEM((T + _L,), jnp.int32),
        pltpu.VMEM((T + _L,), jnp.int32),
        pltpu.VMEM((_WIN + _L,), jnp.int32),
        pltpu.VMEM((_WIN + _L,), jnp.int32),
        pltpu.VMEM((D, _CW), jnp.float32),
        pltpu.VMEM((D, _CW), jnp.float32),
        pltpu.VMEM((_RING * _L, D), jnp.float32),
        pltpu.SemaphoreType.DMA,
        pltpu.SemaphoreType.DMA,
        pltpu.SemaphoreType.DMA,
    ],
    compiler_params=pltpu.CompilerParams(needs_layout_passes=False),
)
def _gather_kernel(idx_hbm, tab_hbm, out_hbm, idx_v, cand_r, cand_p,
                   sel_r, sel_p, buf_a, buf_b, tok_v, sem_a, sem_b, osem):
    wid = lax.axis_index("s") * _NC + lax.axis_index("c")
    lane = lax.iota(jnp.int32, _L)

    bs = wid * _NBLK // _NW
    be = (wid + 1) * _NBLK // _NW
    lo = bs * 128
    hi = be * 128

    pltpu.sync_copy(idx_hbm, idx_v)

    @pl.loop(0, T // _L, init_carry=0)
    def _filter(g, cnt):
        rows = idx_v[pl.ds(g * _L, _L)]
        m = (rows >= lo) & (rows < hi)
        ps = plsc.cumsum(m.astype(jnp.int32))
        offs = jnp.where(m, cnt + ps - 1, T + lane)
        plsc.store_scatter(cand_r, [offs], rows)
        plsc.store_scatter(cand_p, [offs], lane + g * _L)
        return cnt + ps[_L - 1]

    cnt = _filter

    def fetch(blk, buf, sem):
        off = jnp.minimum(blk * 128, _MAXOFF)
        off = pl.multiple_of(off, 128)
        for a in range(D // 8):
            pltpu.async_copy(
                tab_hbm.at[pl.ds(a * 8, 8), pl.ds(off, _CW)],
                buf.at[pl.ds(a * 8, 8), :],
                sem,
            )

    def wait_fetch(buf, sem):
        for a in range(D // 8):
            pltpu.make_async_copy(
                tab_hbm.at[pl.ds(0, 8), pl.ds(0, _CW)],
                buf.at[pl.ds(0, 8), :],
                sem,
            ).wait()

    def process(blk, buf, kgrp, wait_fn):
        clo = blk * 128
        chi = jnp.minimum(blk + _NBC, be) * 128
        off = jnp.minimum(clo, _MAXOFF)

        @pl.loop(0, jnp.maximum((cnt + _WIN - 1) // _WIN, 1), init_carry=kgrp)
        def _window(w, kgrp):
            wbase = w * _WIN
            wlim = jnp.minimum(cnt - wbase, _WIN)

            @pl.loop(0, (wlim + _L - 1) // _L, init_carry=0)
            def _select(gi, scnt):
                rows = cand_r[pl.ds(wbase + gi * _L, _L)]
                pos = cand_p[pl.ds(wbase + gi * _L, _L)]
                m = (rows >= clo) & (rows < chi) & (lane + gi * _L < wlim)
                ps = plsc.cumsum(m.astype(jnp.int32))
                offs = jnp.where(m, scnt + ps - 1, _WIN + lane)
                plsc.store_scatter(sel_r, [offs], rows)
                plsc.store_scatter(sel_p, [offs], pos)
                return scnt + ps[_L - 1]

            scnt = _select

            @pl.when(w == 0)
            def _():
                wait_fn()

            @pl.loop(0, (scnt + _L - 1) // _L, init_carry=kgrp)
            def _groups(t, kgrp):
                rows = sel_r[pl.ds(t * _L, _L)]
                pos = sel_p[pl.ds(t * _L, _L)]
                m = lane < scnt - t * _L
                rows = jnp.where(m, rows, jnp.full((_L,), 1, jnp.int32) * rows[0])
                pos = jnp.where(m, pos, jnp.full((_L,), 1, jnp.int32) * pos[0])
                q = rows - off
                slot = lax.rem(kgrp, jnp.int32(_RING))

                @pl.when(kgrp >= _RING)
                def _():
                    @pl.loop(0, _L)
                    def _dr(j):
                        pltpu.make_async_copy(
                            tok_v.at[pl.ds(0, 1), :],
                            out_hbm.at[pl.ds(0, 1), :],
                            osem,
                        ).wait()

                srow = slot * _L + lane

                @pl.loop(0, D, unroll=8)
                def _cols(c):
                    vals = plsc.load_gather(
                        buf, [jnp.full((_L,), 1, jnp.int32) * c, q])
                    plsc.store_scatter(
                        tok_v, [srow, jnp.full((_L,), 1, jnp.int32) * c], vals)

                for jj in range(_L):
                    p = pos[jj]
                    pltpu.async_copy(
                        tok_v.at[pl.ds(slot * _L + jj, 1), :],
                        out_hbm.at[pl.ds(p, 1), :],
                        osem,
                    )
                return kgrp + 1

            return _groups

        return _window

    fetch(bs, buf_a, sem_a)

    @pl.loop(bs, be, step=2 * _NBC, init_carry=0)
    def _scan(j, kgrp):
        fetch(j + _NBC, buf_b, sem_b)
        kgrp = process(j, buf_a, kgrp, lambda: wait_fetch(buf_a, sem_a))
        fetch(j + 2 * _NBC, buf_a, sem_a)
        kgrp = process(j + _NBC, buf_b, kgrp, lambda: wait_fetch(buf_b, sem_b))
        return kgrp

    kfin = _scan
    wait_fetch(buf_a, sem_a)

    @pl.loop(0, jnp.minimum(kfin, _RING) * _L)
    def _drain(j):
        pltpu.make_async_copy(
            tok_v.at[pl.ds(0, 1), :],
            out_hbm.at[pl.ds(0, 1), :],
            osem,
        ).wait()


def kernel(x, table):
    idx = x.astype(jnp.int32)
    return _gather_kernel(idx, table.T)


# speedup vs baseline: 4.2310x; 1.0843x over previous
import functools

import jax
import jax.numpy as jnp
from jax import lax
from jax.experimental import pallas as pl
from jax.experimental.pallas import tpu as pltpu
from jax.experimental.pallas import tpu_sc as plsc

T = 16384
D = 64
V = 1000000

_info = plsc.get_sparse_core_info()
_NC, _NS = _info.num_cores, _info.num_subcores
_NW = _NC * _NS
_L = 16
_CW = 512
_NBC = _CW // 128
_NBLK = (V + 127) // 128
_MAXOFF = _NBLK * 128 - _CW
_WIN = 2048
_RING = 8

_mesh = plsc.VectorSubcoreMesh(core_axis_name="c", subcore_axis_name="s")


@functools.partial(
    pl.kernel,
    mesh=_mesh,
    out_type=jax.ShapeDtypeStruct((T, D), jnp.float32),
    scratch_types=[
        pltpu.VMEM((T,), jnp.int32),
        pltpu.VMEM((T + _L,), jnp.int32),
        pltpu.VMEM((_WIN + _L,), jnp.int32),
        pltpu.VMEM((_WIN + _L,), jnp.int32),
        pltpu.VMEM((D, _CW), jnp.float32),
        pltpu.VMEM((D, _CW), jnp.float32),
        pltpu.VMEM((_RING * _L, D), jnp.float32),
        pltpu.SemaphoreType.DMA,
        pltpu.SemaphoreType.DMA,
        pltpu.SemaphoreType.DMA,
    ],
    compiler_params=pltpu.CompilerParams(needs_layout_passes=False),
)
def _gather_kernel(idx_hbm, tab_hbm, out_hbm, idx_v, cand_p,
                   sel_r, sel_p, buf_a, buf_b, tok_v, sem_a, sem_b, osem):
    wid = lax.axis_index("s") * _NC + lax.axis_index("c")
    lane = lax.iota(jnp.int32, _L)

    bs = wid * _NBLK // _NW
    be = (wid + 1) * _NBLK // _NW
    lo = bs * 128
    hi = be * 128

    pltpu.sync_copy(idx_hbm, idx_v)

    @pl.loop(0, T // _L, init_carry=0)
    def _filter(g, cnt):
        rows = idx_v[pl.ds(g * _L, _L)]
        m = (rows >= lo) & (rows < hi)
        ps = plsc.cumsum(m.astype(jnp.int32))
        offs = jnp.where(m, cnt + ps - 1, T + lane)
        plsc.store_scatter(cand_p, [offs], lane + g * _L)
        return cnt + ps[_L - 1]

    cnt = _filter

    def fetch(blk, buf, sem):
        off = jnp.minimum(blk * 128, _MAXOFF)
        off = pl.multiple_of(off, 128)
        for a in range(D // 8):
            pltpu.async_copy(
                tab_hbm.at[pl.ds(a * 8, 8), pl.ds(off, _CW)],
                buf.at[pl.ds(a * 8, 8), :],
                sem,
            )

    def wait_fetch(buf, sem):
        for a in range(D // 8):
            pltpu.make_async_copy(
                tab_hbm.at[pl.ds(0, 8), pl.ds(0, _CW)],
                buf.at[pl.ds(0, 8), :],
                sem,
            ).wait()

    def process(blk, buf, kgrp, wait_fn):
        clo = blk * 128
        chi = jnp.minimum(blk + _NBC, be) * 128
        off = jnp.minimum(clo, _MAXOFF)

        @pl.loop(0, jnp.maximum((cnt + _WIN - 1) // _WIN, 1), init_carry=kgrp)
        def _window(w, kgrp):
            wbase = w * _WIN
            wlim = jnp.minimum(cnt - wbase, _WIN)

            @pl.loop(0, (wlim + _L - 1) // _L, init_carry=0)
            def _select(gi, scnt):
                pos = cand_p[pl.ds(wbase + gi * _L, _L)]
                rows = plsc.load_gather(idx_v, [pos & (T - 1)])
                m = (rows >= clo) & (rows < chi) & (lane + gi * _L < wlim)
                ps = plsc.cumsum(m.astype(jnp.int32))
                offs = jnp.where(m, scnt + ps - 1, _WIN + lane)
                plsc.store_scatter(sel_r, [offs], rows)
                plsc.store_scatter(sel_p, [offs], pos)
                return scnt + ps[_L - 1]

            scnt = _select

            @pl.when(w == 0)
            def _():
                wait_fn()

            @pl.loop(0, (scnt + _L - 1) // _L, init_carry=kgrp)
            def _groups(t, kgrp):
                rows = sel_r[pl.ds(t * _L, _L)]
                pos = sel_p[pl.ds(t * _L, _L)]
                m = lane < scnt - t * _L
                rows = jnp.where(m, rows, jnp.full((_L,), 1, jnp.int32) * rows[0])
                pos = jnp.where(m, pos, jnp.full((_L,), 1, jnp.int32) * pos[0])
                q = rows - off
                slot = lax.rem(kgrp, jnp.int32(_RING))

                @pl.when(kgrp >= _RING)
                def _():
                    @pl.loop(0, _L)
                    def _dr(j):
                        pltpu.make_async_copy(
                            tok_v.at[pl.ds(0, 1), :],
                            out_hbm.at[pl.ds(0, 1), :],
                            osem,
                        ).wait()

                srow = slot * _L + lane

                @pl.loop(0, D, unroll=8)
                def _cols(c):
                    vals = plsc.load_gather(
                        buf, [jnp.full((_L,), 1, jnp.int32) * c, q])
                    plsc.store_scatter(
                        tok_v, [srow, jnp.full((_L,), 1, jnp.int32) * c], vals)

                for jj in range(_L):
                    p = pos[jj]
                    pltpu.async_copy(
                        tok_v.at[pl.ds(slot * _L + jj, 1), :],
                        out_hbm.at[pl.ds(p, 1), :],
                        osem,
                    )
                return kgrp + 1

            return _groups

        return _window

    fetch(bs, buf_a, sem_a)

    @pl.loop(bs, be, step=2 * _NBC, init_carry=0)
    def _scan(j, kgrp):
        fetch(j + _NBC, buf_b, sem_b)
        kgrp = process(j, buf_a, kgrp, lambda: wait_fetch(buf_a, sem_a))
        fetch(j + 2 * _NBC, buf_a, sem_a)
        kgrp = process(j + _NBC, buf_b, kgrp, lambda: wait_fetch(buf_b, sem_b))
        return kgrp

    kfin = _scan
    wait_fetch(buf_a, sem_a)

    @pl.loop(0, jnp.minimum(kfin, _RING) * _L)
    def _drain(j):
        pltpu.make_async_copy(
            tok_v.at[pl.ds(0, 1), :],
            out_hbm.at[pl.ds(0, 1), :],
            osem,
        ).wait()


def kernel(x, table):
    idx = x.astype(jnp.int32)
    return _gather_kernel(idx, table.T)


# speedup vs baseline: 4.4602x; 1.0542x over previous
import functools

import jax
import jax.numpy as jnp
from jax import lax
from jax.experimental import pallas as pl
from jax.experimental.pallas import tpu as pltpu
from jax.experimental.pallas import tpu_sc as plsc

T = 16384
D = 64
V = 1000000

_info = plsc.get_sparse_core_info()
_NC, _NS = _info.num_cores, _info.num_subcores
_NW = _NC * _NS
_L = 16
_CW = 640
_NBC = _CW // 128
_NBLK = (V + 127) // 128
_MAXOFF = _NBLK * 128 - _CW
_WIN = 1024
_RING = 4

_mesh = plsc.VectorSubcoreMesh(core_axis_name="c", subcore_axis_name="s")


@functools.partial(
    pl.kernel,
    mesh=_mesh,
    out_type=jax.ShapeDtypeStruct((T, D), jnp.float32),
    scratch_types=[
        pltpu.VMEM((T,), jnp.int32),
        pltpu.VMEM((T + _L,), jnp.int32),
        pltpu.VMEM((_WIN + _L,), jnp.int32),
        pltpu.VMEM((_WIN + _L,), jnp.int32),
        pltpu.VMEM((D, _CW), jnp.float32),
        pltpu.VMEM((D, _CW), jnp.float32),
        pltpu.VMEM((_RING * _L, D), jnp.float32),
        pltpu.SemaphoreType.DMA,
        pltpu.SemaphoreType.DMA,
        pltpu.SemaphoreType.DMA,
    ],
    compiler_params=pltpu.CompilerParams(needs_layout_passes=False),
)
def _gather_kernel(idx_hbm, tab_hbm, out_hbm, idx_v, cand_p,
                   sel_r, sel_p, buf_a, buf_b, tok_v, sem_a, sem_b, osem):
    wid = lax.axis_index("s") * _NC + lax.axis_index("c")
    lane = lax.iota(jnp.int32, _L)

    bs = wid * _NBLK // _NW
    be = (wid + 1) * _NBLK // _NW
    lo = bs * 128
    hi = be * 128

    def fetch(blk, buf, sem):
        off = jnp.minimum(blk * 128, _MAXOFF)
        off = pl.multiple_of(off, 128)
        for a in range(D // 8):
            pltpu.async_copy(
                tab_hbm.at[pl.ds(a * 8, 8), pl.ds(off, _CW)],
                buf.at[pl.ds(a * 8, 8), :],
                sem,
            )

    def wait_fetch(buf, sem):
        for a in range(D // 8):
            pltpu.make_async_copy(
                tab_hbm.at[pl.ds(0, 8), pl.ds(0, _CW)],
                buf.at[pl.ds(0, 8), :],
                sem,
            ).wait()

    pltpu.sync_copy(idx_hbm, idx_v)
    fetch(bs, buf_a, sem_a)

    @pl.loop(0, T // _L, init_carry=0)
    def _filter(g, cnt):
        rows = idx_v[pl.ds(g * _L, _L)]
        m = (rows >= lo) & (rows < hi)
        ps = plsc.cumsum(m.astype(jnp.int32))
        offs = jnp.where(m, cnt + ps - 1, T + lane)
        plsc.store_scatter(cand_p, [offs], lane + g * _L)
        return cnt + ps[_L - 1]

    cnt = _filter

    def process(blk, buf, kgrp, wait_fn):
        clo = blk * 128
        chi = jnp.minimum(blk + _NBC, be) * 128
        off = jnp.minimum(clo, _MAXOFF)

        @pl.loop(0, jnp.maximum((cnt + _WIN - 1) // _WIN, 1), init_carry=kgrp)
        def _window(w, kgrp):
            wbase = w * _WIN
            wlim = jnp.minimum(cnt - wbase, _WIN)

            @pl.loop(0, (wlim + _L - 1) // _L, init_carry=0)
            def _select(gi, scnt):
                pos = cand_p[pl.ds(wbase + gi * _L, _L)]
                rows = plsc.load_gather(idx_v, [pos & (T - 1)])
                m = (rows >= clo) & (rows < chi) & (lane + gi * _L < wlim)
                ps = plsc.cumsum(m.astype(jnp.int32))
                offs = jnp.where(m, scnt + ps - 1, _WIN + lane)
                plsc.store_scatter(sel_r, [offs], rows)
                plsc.store_scatter(sel_p, [offs], pos)
                return scnt + ps[_L - 1]

            scnt = _select

            @pl.when(w == 0)
            def _():
                wait_fn()

            @pl.loop(0, (scnt + _L - 1) // _L, init_carry=kgrp)
            def _groups(t, kgrp):
                rows = sel_r[pl.ds(t * _L, _L)]
                pos = sel_p[pl.ds(t * _L, _L)]
                m = lane < scnt - t * _L
                rows = jnp.where(m, rows, jnp.full((_L,), 1, jnp.int32) * rows[0])
                pos = jnp.where(m, pos, jnp.full((_L,), 1, jnp.int32) * pos[0])
                q = rows - off
                slot = lax.rem(kgrp, jnp.int32(_RING))

                @pl.when(kgrp >= _RING)
                def _():
                    @pl.loop(0, _L)
                    def _dr(j):
                        pltpu.make_async_copy(
                            tok_v.at[pl.ds(0, 1), :],
                            out_hbm.at[pl.ds(0, 1), :],
                            osem,
                        ).wait()

                srow = slot * _L + lane

                @pl.loop(0, D, unroll=8)
                def _cols(c):
                    vals = plsc.load_gather(
                        buf, [jnp.full((_L,), 1, jnp.int32) * c, q])
                    plsc.store_scatter(
                        tok_v, [srow, jnp.full((_L,), 1, jnp.int32) * c], vals)

                for jj in range(_L):
                    p = pos[jj]
                    pltpu.async_copy(
                        tok_v.at[pl.ds(slot * _L + jj, 1), :],
                        out_hbm.at[pl.ds(p, 1), :],
                        osem,
                    )
                return kgrp + 1

            return _groups

        return _window

    @pl.loop(bs, be, step=2 * _NBC, init_carry=0)
    def _scan(j, kgrp):
        fetch(j + _NBC, buf_b, sem_b)
        kgrp = process(j, buf_a, kgrp, lambda: wait_fetch(buf_a, sem_a))
        fetch(j + 2 * _NBC, buf_a, sem_a)
        kgrp = process(j + _NBC, buf_b, kgrp, lambda: wait_fetch(buf_b, sem_b))
        return kgrp

    kfin = _scan
    wait_fetch(buf_a, sem_a)

    @pl.loop(0, jnp.minimum(kfin, _RING) * _L)
    def _drain(j):
        pltpu.make_async_copy(
            tok_v.at[pl.ds(0, 1), :],
            out_hbm.at[pl.ds(0, 1), :],
            osem,
        ).wait()


def kernel(x, table):
    idx = x.astype(jnp.int32)
    return _gather_kernel(idx, table.T)
